# baseline (speedup 1.0000x reference)
import functools

import jax
import jax.numpy as jnp
from jax import lax
from jax.experimental import pallas as pl
from jax.experimental.pallas import tpu as pltpu
from jax.experimental.pallas import tpu_sc as plsc

T, N, E, F, H = 4, 10000, 320000, 128, 128
P = 10240
TP = T * P
NC, NS = 2, 16
NW = NC * NS
B = 64
EW = 10240
NCHUNK = EW // B
NQ = 4
QCH = NCHUNK // NQ
QCH2 = QCH // 2
EPAD = EW * NW
RT = P // NS


def _sc_mesh():
  return plsc.VectorSubcoreMesh(core_axis_name="c", subcore_axis_name="s")


def _deg_body(dst_hbm, ones_hbm, zeros_hbm, out_hbm, acc, ones_b,
              dstblk, sem0, sem1, sem2, sem3):
  c = lax.axis_index("c")
  s = lax.axis_index("s")
  w = c * NS + s
  pltpu.sync_copy(zeros_hbm, acc.at[pl.ds(s * RT, RT)])
  plsc.subcore_barrier()
  sems = (sem0, sem1, sem2, sem3)
  for t in range(T):
    pltpu.sync_copy(ones_hbm.at[t], ones_b)
    for q in range(NQ):
      rowbase = (t * NW + w) * NCHUNK + q * QCH
      pltpu.sync_copy(dst_hbm.at[pl.ds(rowbase, QCH)], dstblk)
      def chunk(k, carry):
        ds = [pltpu.async_copy(ones_b, acc.at[dstblk.at[4 * k + u]], sems[u],
                               add=True) for u in range(4)]
        for d in ds:
          d.wait()
        return carry
      lax.fori_loop(0, QCH // 4, chunk, 0)
  plsc.subcore_barrier()
  pltpu.sync_copy(acc.at[pl.ds(s * RT, RT)], out_hbm.at[c, pl.ds(s * RT, RT)])


def _sc_deg(dst_flat):
  k = pl.kernel(
      _deg_body,
      out_type=jax.ShapeDtypeStruct((NC, P, H), jnp.float32),
      mesh=_sc_mesh(),
      scratch_types=[
          pltpu.VMEM_SHARED((P, H), jnp.float32),
          pltpu.VMEM((B, H), jnp.float32),
          pltpu.VMEM((QCH, B), jnp.int32),
          pltpu.SemaphoreType.DMA,
          pltpu.SemaphoreType.DMA,
          pltpu.SemaphoreType.DMA,
          pltpu.SemaphoreType.DMA,
      ],
  )
  tsel = (jnp.arange(H)[None, :] // 32 == jnp.arange(T)[:, None])
  ones4 = jnp.broadcast_to(tsel.astype(jnp.float32)[:, None, :], (T, B, H))
  return k(dst_flat.reshape(-1, B), ones4, jnp.zeros((RT, H), jnp.float32))


def _agg_body(y_hbm, src_hbm, dst_hbm, zeros_hbm, out_hbm, acc, srcblk,
              dstblk, rows0, rows1, gsem0, gsem1, ssem0, ssem1):
  c = lax.axis_index("c")
  s = lax.axis_index("s")
  w = c * NS + s

  def g_start(j, rbuf, sem):
    pltpu.async_copy(y_hbm.at[srcblk.at[j]], rbuf, sem)

  def g_wait(rbuf, sem):
    pltpu.make_async_copy(y_hbm.at[srcblk.at[0]], rbuf, sem).wait()

  pltpu.sync_copy(zeros_hbm, acc.at[pl.ds(s * RT, RT)])
  plsc.subcore_barrier()
  for q in range(NQ):
    rowbase = w * NCHUNK + q * QCH
    pltpu.sync_copy(src_hbm.at[pl.ds(rowbase, QCH)], srcblk)
    pltpu.sync_copy(dst_hbm.at[pl.ds(rowbase, QCH)], dstblk)
    g_start(0, rows0, gsem0)
    g_start(1, rows1, gsem1)
    def chunk2(jj, carry):
      j0 = 2 * jj
      more = jj < QCH2 - 1
      g_wait(rows0, gsem0)
      sd0 = pltpu.async_copy(rows0, acc.at[dstblk.at[j0]], ssem0, add=True)
      g_wait(rows1, gsem1)
      sd0.wait()
      @pl.when(more)
      def _():
        g_start(j0 + 2, rows0, gsem0)
      sd1 = pltpu.async_copy(rows1, acc.at[dstblk.at[j0 + 1]], ssem1,
                             add=True)
      sd1.wait()
      @pl.when(more)
      def _():
        g_start(j0 + 3, rows1, gsem1)
      return carry
    lax.fori_loop(0, QCH2, chunk2, 0)
  plsc.subcore_barrier()
  pltpu.sync_copy(acc.at[pl.ds(s * RT, RT)], out_hbm.at[c, pl.ds(s * RT, RT)])


def _sc_agg(y_t, src2d_t, dst2d_t, zeros):
  k = pl.kernel(
      _agg_body,
      out_type=jax.ShapeDtypeStruct((NC, P, H), jnp.float32),
      mesh=_sc_mesh(),
      scratch_types=[
          pltpu.VMEM_SHARED((P, H), jnp.float32),
          pltpu.VMEM((QCH, B), jnp.int32),
          pltpu.VMEM((QCH, B), jnp.int32),
          pltpu.VMEM((B, H), jnp.float32),
          pltpu.VMEM((B, H), jnp.float32),
          pltpu.SemaphoreType.DMA,
          pltpu.SemaphoreType.DMA,
          pltpu.SemaphoreType.DMA,
          pltpu.SemaphoreType.DMA,
      ],
  )
  return k(y_t, src2d_t, dst2d_t, zeros)


def _xw_body(x_ref, w_ref, out_ref):
  out_ref[...] = jnp.dot(x_ref[...], w_ref[...],
                         preferred_element_type=jnp.float32)


def _tc_xw(x, w1, blk=2048):
  return pl.pallas_call(
      _xw_body,
      grid=(TP // blk,),
      in_specs=[
          pl.BlockSpec((blk, F), lambda i: (i, 0)),
          pl.BlockSpec((F, H), lambda i: (0, 0)),
      ],
      out_specs=pl.BlockSpec((blk, H), lambda i: (i, 0)),
      out_shape=jax.ShapeDtypeStruct((TP, H), jnp.float32),
  )(x, w1)


def _y1s_body(degp_ref, xw_ref, y1_ref, dinv_ref):
  d = degp_ref[0] + degp_ref[1]
  dv = lax.rsqrt(jnp.maximum(d + 1.0, 1.0))
  t = pl.program_id(0)
  lanes = lax.broadcasted_iota(jnp.int32, (1, H), 1)
  mask = (lanes == 32 * t).astype(jnp.float32)
  dcol = jnp.sum(dv * mask, axis=1, keepdims=True)
  y1_ref[...] = xw_ref[...] * dcol
  dinv_ref[0] = dcol


def _tc_y1s(degp, xw, blk=2048):
  nb = P // blk
  return pl.pallas_call(
      _y1s_body,
      grid=(T, nb),
      in_specs=[
          pl.BlockSpec((NC, blk, H), lambda t, i: (0, i, 0)),
          pl.BlockSpec((blk, H), lambda t, i: (t * nb + i, 0)),
      ],
      out_specs=[
          pl.BlockSpec((blk, H), lambda t, i: (t * nb + i, 0)),
          pl.BlockSpec((1, blk, 1), lambda t, i: (t, i, 0)),
      ],
      out_shape=[
          jax.ShapeDtypeStruct((TP, H), jnp.float32),
          jax.ShapeDtypeStruct((T, P, 1), jnp.float32),
      ],
  )(degp, xw)


def _y2_body(agg_ref, y1_ref, dinv_ref, b1_ref, w2_ref, out_ref):
  dv = dinv_ref[0]
  a = agg_ref[0] + agg_ref[1] + y1_ref[...]
  h1 = jax.nn.relu(dv * a + b1_ref[...])
  out_ref[...] = jnp.dot(h1, w2_ref[...],
                         preferred_element_type=jnp.float32) * dv


def _tc_y2(aggp_t, y1, dinv, b1, w2, t, blk=2048):
  nb = P // blk
  return pl.pallas_call(
      _y2_body,
      grid=(nb,),
      in_specs=[
          pl.BlockSpec((NC, blk, H), lambda i: (0, i, 0)),
          pl.BlockSpec((blk, H), lambda i, t=t: (t * nb + i, 0)),
          pl.BlockSpec((1, blk, 1), lambda i, t=t: (t, i, 0)),
          pl.BlockSpec((1, H), lambda i: (0, 0)),
          pl.BlockSpec((H, H), lambda i: (0, 0)),
      ],
      out_specs=pl.BlockSpec((blk, H), lambda i: (i, 0)),
      out_shape=jax.ShapeDtypeStruct((P, H), jnp.float32),
  )(aggp_t, y1, dinv, b1, w2)


def _lstm_body(*refs):
  aggs = refs[0:T]
  y2s = refs[T:2 * T]
  dinv_ref, b2_ref, wih_ref, whh_ref, bg_ref, out_ref = refs[2 * T:]
  blk = out_ref.shape[0]
  h = jnp.zeros((blk, H), jnp.float32)
  c = jnp.zeros((blk, H), jnp.float32)
  for t in range(T):
    a = aggs[t][0] + aggs[t][1] + y2s[t][...]
    h2 = jax.nn.relu(dinv_ref[t] * a + b2_ref[...])
    g = (jnp.dot(h2, wih_ref[...], preferred_element_type=jnp.float32)
         + jnp.dot(h, whh_ref[...], preferred_element_type=jnp.float32)
         + bg_ref[...])
    i_g = jax.nn.sigmoid(g[:, 0 * H:1 * H])
    f_g = jax.nn.sigmoid(g[:, 1 * H:2 * H])
    g_g = jnp.tanh(g[:, 2 * H:3 * H])
    o_g = jax.nn.sigmoid(g[:, 3 * H:4 * H])
    c = f_g * c + i_g * g_g
    h = o_g * jnp.tanh(c)
  out_ref[...] = h


def _tc_lstm(aggs, y2s, dinv, b2, wih_t, whh_t, bg, blk=2048):
  return pl.pallas_call(
      _lstm_body,
      grid=(P // blk,),
      in_specs=(
          [pl.BlockSpec((NC, blk, H), lambda i: (0, i, 0)) for _ in range(T)]
          + [pl.BlockSpec((blk, H), lambda i: (i, 0)) for _ in range(T)]
          + [
              pl.BlockSpec((T, blk, 1), lambda i: (0, i, 0)),
              pl.BlockSpec((1, H), lambda i: (0, 0)),
              pl.BlockSpec((H, 4 * H), lambda i: (0, 0)),
              pl.BlockSpec((H, 4 * H), lambda i: (0, 0)),
              pl.BlockSpec((1, 4 * H), lambda i: (0, 0)),
          ]),
      out_specs=pl.BlockSpec((blk, H), lambda i: (i, 0)),
      out_shape=jax.ShapeDtypeStruct((P, H), jnp.float32),
  )(*aggs, *y2s, dinv, b2, wih_t, whh_t, bg)


def kernel(node_features_seq, edge_indices_seq, W1, b1, W2, b2,
           W_ih, W_hh, b_ih, b_hh):
  x = jnp.pad(node_features_seq, ((0, 0), (0, P - N), (0, 0)))
  pad_src = (jnp.arange(EPAD - E) % N).astype(jnp.int32)
  pad_dst = (N + (jnp.arange(EPAD - E) % (P - N))).astype(jnp.int32)
  src = jnp.concatenate(
      [edge_indices_seq[:, 0, :],
       jnp.broadcast_to(pad_src, (T, EPAD - E))], axis=1)
  dst = jnp.concatenate(
      [edge_indices_seq[:, 1, :],
       jnp.broadcast_to(pad_dst, (T, EPAD - E))], axis=1)
  src2d = src.reshape(T, EPAD // B, B)
  dst2d = dst.reshape(T, EPAD // B, B)
  zeros = jnp.zeros((RT, H), jnp.float32)

  degp = _sc_deg(dst.reshape(-1))
  xw = _tc_xw(x.reshape(TP, F), W1)
  y1, dinv3 = _tc_y1s(degp, xw)
  y1t = y1.reshape(T, P, H)

  agg1 = [_sc_agg(y1t[t], src2d[t], dst2d[t], zeros) for t in range(T)]
  b1r = b1.reshape(1, H)
  y2s = []
  agg2 = []
  for t in range(T):
    y2_t = _tc_y2(agg1[t], y1, dinv3, b1r, W2, t)
    y2s.append(y2_t)
    agg2.append(_sc_agg(y2_t, src2d[t], dst2d[t], zeros))

  h = _tc_lstm(agg2, y2s, dinv3, b2.reshape(1, H), W_ih.T, W_hh.T,
               (b_ih + b_hh).reshape(1, 4 * H))
  return h[:N]


# speedup vs baseline: 1.0220x; 1.0220x over previous
import functools

import jax
import jax.numpy as jnp
from jax import lax
from jax.experimental import pallas as pl
from jax.experimental.pallas import tpu as pltpu
from jax.experimental.pallas import tpu_sc as plsc

T, N, E, F, H = 4, 10000, 320000, 128, 128
P = 10240
TP = T * P
NC, NS = 2, 16
NW = NC * NS
B = 64
EW = 10240
NCHUNK = EW // B
NQ = 4
QCH = NCHUNK // NQ
QCH2 = QCH // 2
EPAD = EW * NW
RT = P // NS


def _sc_mesh():
  return plsc.VectorSubcoreMesh(core_axis_name="c", subcore_axis_name="s")


def _deg_body(dst_hbm, ones_hbm, zeros_hbm, out_hbm, acc, ones_b,
              dstblk, sem0, sem1, sem2, sem3):
  c = lax.axis_index("c")
  s = lax.axis_index("s")
  w = c * NS + s
  pltpu.sync_copy(zeros_hbm, acc.at[pl.ds(s * RT, RT)])
  plsc.subcore_barrier()
  sems = (sem0, sem1, sem2, sem3)
  for t in range(T):
    pltpu.sync_copy(ones_hbm.at[t], ones_b)
    for q in range(NQ):
      rowbase = (t * NW + w) * NCHUNK + q * QCH
      pltpu.sync_copy(dst_hbm.at[pl.ds(rowbase, QCH)], dstblk)
      def chunk(k, carry):
        ds = [pltpu.async_copy(ones_b, acc.at[dstblk.at[4 * k + u]], sems[u],
                               add=True) for u in range(4)]
        for d in ds:
          d.wait()
        return carry
      lax.fori_loop(0, QCH // 4, chunk, 0)
  plsc.subcore_barrier()
  pltpu.sync_copy(acc.at[pl.ds(s * RT, RT)], out_hbm.at[c, pl.ds(s * RT, RT)])


def _sc_deg(dst_flat):
  k = pl.kernel(
      _deg_body,
      out_type=jax.ShapeDtypeStruct((NC, P, H), jnp.float32),
      mesh=_sc_mesh(),
      scratch_types=[
          pltpu.VMEM_SHARED((P, H), jnp.float32),
          pltpu.VMEM((B, H), jnp.float32),
          pltpu.VMEM((QCH, B), jnp.int32),
          pltpu.SemaphoreType.DMA,
          pltpu.SemaphoreType.DMA,
          pltpu.SemaphoreType.DMA,
          pltpu.SemaphoreType.DMA,
      ],
  )
  tsel = (jnp.arange(H)[None, :] // 32 == jnp.arange(T)[:, None])
  ones4 = jnp.broadcast_to(tsel.astype(jnp.float32)[:, None, :], (T, B, H))
  return k(dst_flat.reshape(-1, B), ones4, jnp.zeros((RT, H), jnp.float32))


def _agg_body(y_hbm, src_hbm, dst_hbm, zeros_hbm, out_hbm, acc, srcblk,
              dstblk, rows0, rows1, gsem0, gsem1, ssem0, ssem1):
  c = lax.axis_index("c")
  s = lax.axis_index("s")
  w = c * NS + s

  def g_start(j, rbuf, sem):
    pltpu.async_copy(y_hbm.at[srcblk.at[j]], rbuf, sem)

  def g_wait(rbuf, sem):
    pltpu.make_async_copy(y_hbm.at[srcblk.at[0]], rbuf, sem).wait()

  for t in range(T):
    pltpu.sync_copy(zeros_hbm, acc.at[pl.ds(s * RT, RT)])
    plsc.subcore_barrier()
    yoff = t * P
    for q in range(NQ):
      rowbase = (t * NW + w) * NCHUNK + q * QCH
      pltpu.sync_copy(src_hbm.at[pl.ds(rowbase, QCH)], srcblk)
      pltpu.sync_copy(dst_hbm.at[pl.ds(rowbase, QCH)], dstblk)
      def offrow(j, carry):
        for u in range(B // 16):
          srcblk[j, pl.ds(u * 16, 16)] = srcblk[j, pl.ds(u * 16, 16)] + yoff
        return carry
      lax.fori_loop(0, QCH, offrow, 0)
      g_start(0, rows0, gsem0)
      g_start(1, rows1, gsem1)
      def chunk2(jj, carry):
        j0 = 2 * jj
        more = jj < QCH2 - 1
        g_wait(rows0, gsem0)
        sd0 = pltpu.async_copy(rows0, acc.at[dstblk.at[j0]], ssem0, add=True)
        g_wait(rows1, gsem1)
        sd0.wait()
        @pl.when(more)
        def _():
          g_start(j0 + 2, rows0, gsem0)
        sd1 = pltpu.async_copy(rows1, acc.at[dstblk.at[j0 + 1]], ssem1,
                               add=True)
        sd1.wait()
        @pl.when(more)
        def _():
          g_start(j0 + 3, rows1, gsem1)
        return carry
      lax.fori_loop(0, QCH2, chunk2, 0)
    plsc.subcore_barrier()
    pltpu.sync_copy(acc.at[pl.ds(s * RT, RT)],
                    out_hbm.at[c, t, pl.ds(s * RT, RT)])


def _sc_agg(y_flat, src2d, dst2d, zeros):
  k = pl.kernel(
      _agg_body,
      out_type=jax.ShapeDtypeStruct((NC, T, P, H), jnp.float32),
      mesh=_sc_mesh(),
      scratch_types=[
          pltpu.VMEM_SHARED((P, H), jnp.float32),
          pltpu.VMEM((QCH, B), jnp.int32),
          pltpu.VMEM((QCH, B), jnp.int32),
          pltpu.VMEM((B, H), jnp.float32),
          pltpu.VMEM((B, H), jnp.float32),
          pltpu.SemaphoreType.DMA,
          pltpu.SemaphoreType.DMA,
          pltpu.SemaphoreType.DMA,
          pltpu.SemaphoreType.DMA,
      ],
  )
  return k(y_flat, src2d, dst2d, zeros)


def _xw_body(x_ref, w_ref, out_ref):
  out_ref[...] = jnp.dot(x_ref[...], w_ref[...],
                         preferred_element_type=jnp.float32)


def _tc_xw(x, w1, blk=2048):
  return pl.pallas_call(
      _xw_body,
      grid=(TP // blk,),
      in_specs=[
          pl.BlockSpec((blk, F), lambda i: (i, 0)),
          pl.BlockSpec((F, H), lambda i: (0, 0)),
      ],
      out_specs=pl.BlockSpec((blk, H), lambda i: (i, 0)),
      out_shape=jax.ShapeDtypeStruct((TP, H), jnp.float32),
  )(x, w1)


def _y1s_body(degp_ref, xw_ref, y1_ref, dinv_ref):
  d = degp_ref[0] + degp_ref[1]
  dv = lax.rsqrt(jnp.maximum(d + 1.0, 1.0))
  t = pl.program_id(0)
  lanes = lax.broadcasted_iota(jnp.int32, (1, H), 1)
  mask = (lanes == 32 * t).astype(jnp.float32)
  dcol = jnp.sum(dv * mask, axis=1, keepdims=True)
  y1_ref[...] = xw_ref[...] * dcol
  dinv_ref[0] = dcol


def _tc_y1s(degp, xw, blk=2048):
  nb = P // blk
  return pl.pallas_call(
      _y1s_body,
      grid=(T, nb),
      in_specs=[
          pl.BlockSpec((NC, blk, H), lambda t, i: (0, i, 0)),
          pl.BlockSpec((blk, H), lambda t, i: (t * nb + i, 0)),
      ],
      out_specs=[
          pl.BlockSpec((blk, H), lambda t, i: (t * nb + i, 0)),
          pl.BlockSpec((1, blk, 1), lambda t, i: (t, i, 0)),
      ],
      out_shape=[
          jax.ShapeDtypeStruct((TP, H), jnp.float32),
          jax.ShapeDtypeStruct((T, P, 1), jnp.float32),
      ],
  )(degp, xw)


def _y2_body(agg_ref, y1_ref, dinv_ref, b1_ref, w2_ref, out_ref):
  dv = dinv_ref[0]
  a = agg_ref[0] + agg_ref[1] + y1_ref[...]
  h1 = jax.nn.relu(dv * a + b1_ref[...])
  out_ref[...] = jnp.dot(h1, w2_ref[...],
                         preferred_element_type=jnp.float32) * dv


def _tc_y2(aggp, y1, dinv, b1, w2, blk=2048):
  return pl.pallas_call(
      _y2_body,
      grid=(TP // blk,),
      in_specs=[
          pl.BlockSpec((NC, blk, H), lambda i: (0, i, 0)),
          pl.BlockSpec((blk, H), lambda i: (i, 0)),
          pl.BlockSpec((1, blk, 1), lambda i: (0, i, 0)),
          pl.BlockSpec((1, H), lambda i: (0, 0)),
          pl.BlockSpec((H, H), lambda i: (0, 0)),
      ],
      out_specs=pl.BlockSpec((blk, H), lambda i: (i, 0)),
      out_shape=jax.ShapeDtypeStruct((TP, H), jnp.float32),
  )(aggp, y1, dinv, b1, w2)


def _lstm_body(agg_ref, y2_ref, dinv_ref, b2_ref, wih_ref, whh_ref, bg_ref,
               out_ref):
  blk = out_ref.shape[0]
  h = jnp.zeros((blk, H), jnp.float32)
  c = jnp.zeros((blk, H), jnp.float32)
  for t in range(T):
    a = agg_ref[0, t] + agg_ref[1, t] + y2_ref[t]
    h2 = jax.nn.relu(dinv_ref[t] * a + b2_ref[...])
    g = (jnp.dot(h2, wih_ref[...], preferred_element_type=jnp.float32)
         + jnp.dot(h, whh_ref[...], preferred_element_type=jnp.float32)
         + bg_ref[...])
    i_g = jax.nn.sigmoid(g[:, 0 * H:1 * H])
    f_g = jax.nn.sigmoid(g[:, 1 * H:2 * H])
    g_g = jnp.tanh(g[:, 2 * H:3 * H])
    o_g = jax.nn.sigmoid(g[:, 3 * H:4 * H])
    c = f_g * c + i_g * g_g
    h = o_g * jnp.tanh(c)
  out_ref[...] = h


def _tc_lstm(aggp, y2, dinv, b2, wih_t, whh_t, bg, blk=2048):
  return pl.pallas_call(
      _lstm_body,
      grid=(P // blk,),
      in_specs=[
          pl.BlockSpec((NC, T, blk, H), lambda i: (0, 0, i, 0)),
          pl.BlockSpec((T, blk, H), lambda i: (0, i, 0)),
          pl.BlockSpec((T, blk, 1), lambda i: (0, i, 0)),
          pl.BlockSpec((1, H), lambda i: (0, 0)),
          pl.BlockSpec((H, 4 * H), lambda i: (0, 0)),
          pl.BlockSpec((H, 4 * H), lambda i: (0, 0)),
          pl.BlockSpec((1, 4 * H), lambda i: (0, 0)),
      ],
      out_specs=pl.BlockSpec((blk, H), lambda i: (i, 0)),
      out_shape=jax.ShapeDtypeStruct((P, H), jnp.float32),
  )(aggp, y2, dinv, b2, wih_t, whh_t, bg)


def kernel(node_features_seq, edge_indices_seq, W1, b1, W2, b2,
           W_ih, W_hh, b_ih, b_hh):
  x = jnp.pad(node_features_seq, ((0, 0), (0, P - N), (0, 0)))
  pad_src = (jnp.arange(EPAD - E) % N).astype(jnp.int32)
  pad_dst = (N + (jnp.arange(EPAD - E) % (P - N))).astype(jnp.int32)
  src = jnp.concatenate(
      [edge_indices_seq[:, 0, :],
       jnp.broadcast_to(pad_src, (T, EPAD - E))], axis=1)
  dst = jnp.concatenate(
      [edge_indices_seq[:, 1, :],
       jnp.broadcast_to(pad_dst, (T, EPAD - E))], axis=1)
  src2d = src.reshape(-1, B)
  dst2d = dst.reshape(-1, B)
  zeros = jnp.zeros((RT, H), jnp.float32)

  degp = _sc_deg(dst.reshape(-1))
  xw = _tc_xw(x.reshape(TP, F), W1)
  y1, dinv3 = _tc_y1s(degp, xw)
  dinv_tp = dinv3.reshape(1, TP, 1)

  agg1 = _sc_agg(y1, src2d, dst2d, zeros)
  y2 = _tc_y2(agg1.reshape(NC, TP, H), y1, dinv_tp, b1.reshape(1, H), W2)
  agg2 = _sc_agg(y2, src2d, dst2d, zeros)

  h = _tc_lstm(agg2, y2.reshape(T, P, H), dinv3, b2.reshape(1, H),
               W_ih.T, W_hh.T, (b_ih + b_hh).reshape(1, 4 * H))
  return h[:N]


# speedup vs baseline: 1.0236x; 1.0016x over previous
import functools

import jax
import jax.numpy as jnp
from jax import lax
from jax.experimental import pallas as pl
from jax.experimental.pallas import tpu as pltpu
from jax.experimental.pallas import tpu_sc as plsc

T, N, E, F, H = 4, 10000, 320000, 128, 128
P = 10240
TP = T * P
NC, NS = 2, 16
NW = NC * NS
B = 64
EW = 10240
NCHUNK = EW // B
NQ = 4
QCH = NCHUNK // NQ
QCH2 = QCH // 2
EPAD = EW * NW
RT = P // NS
DW = 16


def _sc_mesh():
  return plsc.VectorSubcoreMesh(core_axis_name="c", subcore_axis_name="s")


def _deg_body(dst_hbm, ones_hbm, zeros_hbm, out_hbm, acc, ones_b,
              dstblk, sem0, sem1, sem2, sem3):
  c = lax.axis_index("c")
  s = lax.axis_index("s")
  w = c * NS + s
  pltpu.sync_copy(zeros_hbm, acc.at[pl.ds(s * RT, RT)])
  plsc.subcore_barrier()
  sems = (sem0, sem1, sem2, sem3)
  for t in range(T):
    pltpu.sync_copy(ones_hbm.at[t], ones_b)
    for q in range(NQ):
      rowbase = (t * NW + w) * NCHUNK + q * QCH
      pltpu.sync_copy(dst_hbm.at[pl.ds(rowbase, QCH)], dstblk)
      def chunk(k, carry):
        ds = [pltpu.async_copy(ones_b, acc.at[dstblk.at[4 * k + u]], sems[u],
                               add=True) for u in range(4)]
        for d in ds:
          d.wait()
        return carry
      lax.fori_loop(0, QCH // 4, chunk, 0)
  plsc.subcore_barrier()
  pltpu.sync_copy(acc.at[pl.ds(s * RT, RT)], out_hbm.at[c, pl.ds(s * RT, RT)])


def _sc_deg(dst_flat):
  k = pl.kernel(
      _deg_body,
      out_type=jax.ShapeDtypeStruct((NC, P, H), jnp.float32),
      mesh=_sc_mesh(),
      scratch_types=[
          pltpu.VMEM_SHARED((P, H), jnp.float32),
          pltpu.VMEM((B, H), jnp.float32),
          pltpu.VMEM((QCH, B), jnp.int32),
          pltpu.SemaphoreType.DMA,
          pltpu.SemaphoreType.DMA,
          pltpu.SemaphoreType.DMA,
          pltpu.SemaphoreType.DMA,
      ],
  )
  tsel = (jnp.arange(H)[None, :] // 32 == jnp.arange(T)[:, None])
  ones4 = jnp.broadcast_to(tsel.astype(jnp.float32)[:, None, :], (T, B, H))
  return k(dst_flat.reshape(-1, B), ones4, jnp.zeros((RT, H), jnp.float32))


def _agg_body(y_hbm, src_hbm, dst_hbm, zeros_hbm, out_hbm, acc, srcblk,
              dstblk, rows0, rows1, gsem0, gsem1, ssem0, ssem1):
  c = lax.axis_index("c")
  s = lax.axis_index("s")
  w = c * NS + s

  def g_start(j, rbuf, sem):
    pltpu.async_copy(y_hbm.at[srcblk.at[j]], rbuf, sem)

  def g_wait(rbuf, sem):
    pltpu.make_async_copy(y_hbm.at[srcblk.at[0]], rbuf, sem).wait()

  for t in range(T):
    pltpu.sync_copy(zeros_hbm, acc.at[pl.ds(s * RT, RT)])
    plsc.subcore_barrier()
    yoff = t * P
    for q in range(NQ):
      rowbase = (t * NW + w) * NCHUNK + q * QCH
      pltpu.sync_copy(src_hbm.at[pl.ds(rowbase, QCH)], srcblk)
      pltpu.sync_copy(dst_hbm.at[pl.ds(rowbase, QCH)], dstblk)
      def offrow(j, carry):
        for u in range(B // 16):
          srcblk[j, pl.ds(u * 16, 16)] = srcblk[j, pl.ds(u * 16, 16)] + yoff
        return carry
      lax.fori_loop(0, QCH, offrow, 0)
      g_start(0, rows0, gsem0)
      g_start(1, rows1, gsem1)
      def chunk2(jj, carry):
        j0 = 2 * jj
        more = jj < QCH2 - 1
        g_wait(rows0, gsem0)
        sd0 = pltpu.async_copy(rows0, acc.at[dstblk.at[j0]], ssem0, add=True)
        g_wait(rows1, gsem1)
        sd0.wait()
        @pl.when(more)
        def _():
          g_start(j0 + 2, rows0, gsem0)
        sd1 = pltpu.async_copy(rows1, acc.at[dstblk.at[j0 + 1]], ssem1,
                               add=True)
        sd1.wait()
        @pl.when(more)
        def _():
          g_start(j0 + 3, rows1, gsem1)
        return carry
      lax.fori_loop(0, QCH2, chunk2, 0)
    plsc.subcore_barrier()
    pltpu.sync_copy(acc.at[pl.ds(s * RT, RT)],
                    out_hbm.at[c, t, pl.ds(s * RT, RT)])


def _sc_agg(y_flat, src2d, dst2d, zeros):
  k = pl.kernel(
      _agg_body,
      out_type=jax.ShapeDtypeStruct((NC, T, P, H), jnp.float32),
      mesh=_sc_mesh(),
      scratch_types=[
          pltpu.VMEM_SHARED((P, H), jnp.float32),
          pltpu.VMEM((QCH, B), jnp.int32),
          pltpu.VMEM((QCH, B), jnp.int32),
          pltpu.VMEM((B, H), jnp.float32),
          pltpu.VMEM((B, H), jnp.float32),
          pltpu.SemaphoreType.DMA,
          pltpu.SemaphoreType.DMA,
          pltpu.SemaphoreType.DMA,
          pltpu.SemaphoreType.DMA,
      ],
  )
  return k(y_flat, src2d, dst2d, zeros)


def _xw_body(x_ref, w_ref, out_ref):
  out_ref[...] = jnp.dot(x_ref[...], w_ref[...],
                         preferred_element_type=jnp.float32)


def _tc_xw(x, w1, blk=2048):
  return pl.pallas_call(
      _xw_body,
      grid=(TP // blk,),
      in_specs=[
          pl.BlockSpec((blk, F), lambda i: (i, 0)),
          pl.BlockSpec((F, H), lambda i: (0, 0)),
      ],
      out_specs=pl.BlockSpec((blk, H), lambda i: (i, 0)),
      out_shape=jax.ShapeDtypeStruct((TP, H), jnp.float32),
  )(x, w1)


def _y1s_body(degp_ref, xw_ref, y1_ref, dinv_ref):
  d = degp_ref[0] + degp_ref[1]
  dv = lax.rsqrt(jnp.maximum(d + 1.0, 1.0))
  t = pl.program_id(0)
  lanes = lax.broadcasted_iota(jnp.int32, (1, H), 1)
  mask = (lanes == 32 * t).astype(jnp.float32)
  dcol = jnp.sum(dv * mask, axis=1, keepdims=True)
  y1_ref[...] = xw_ref[...] * dcol
  dinv_ref[0] = dcol


def _tc_y1s(degp, xw, blk=2048):
  nb = P // blk
  return pl.pallas_call(
      _y1s_body,
      grid=(T, nb),
      in_specs=[
          pl.BlockSpec((NC, blk, H), lambda t, i: (0, i, 0)),
          pl.BlockSpec((blk, H), lambda t, i: (t * nb + i, 0)),
      ],
      out_specs=[
          pl.BlockSpec((blk, H), lambda t, i: (t * nb + i, 0)),
          pl.BlockSpec((1, blk, 1), lambda t, i: (t, i, 0)),
      ],
      out_shape=[
          jax.ShapeDtypeStruct((TP, H), jnp.float32),
          jax.ShapeDtypeStruct((T, P, 1), jnp.float32),
      ],
  )(degp, xw)


def _y2_body(agg_ref, y1_ref, dinv_ref, b1_ref, w2_ref, out_ref):
  dv = dinv_ref[0]
  a = agg_ref[0] + agg_ref[1] + y1_ref[...]
  h1 = jax.nn.relu(dv * a + b1_ref[...])
  out_ref[...] = jnp.dot(h1, w2_ref[...],
                         preferred_element_type=jnp.float32) * dv


def _tc_y2(aggp, y1, dinv, b1, w2, blk=2048):
  return pl.pallas_call(
      _y2_body,
      grid=(TP // blk,),
      in_specs=[
          pl.BlockSpec((NC, blk, H), lambda i: (0, i, 0)),
          pl.BlockSpec((blk, H), lambda i: (i, 0)),
          pl.BlockSpec((1, blk, 1), lambda i: (0, i, 0)),
          pl.BlockSpec((1, H), lambda i: (0, 0)),
          pl.BlockSpec((H, H), lambda i: (0, 0)),
      ],
      out_specs=pl.BlockSpec((blk, H), lambda i: (i, 0)),
      out_shape=jax.ShapeDtypeStruct((TP, H), jnp.float32),
  )(aggp, y1, dinv, b1, w2)


def _lstm_body(agg_ref, y2_ref, dinv_ref, b2_ref, wih_ref, whh_ref, bg_ref,
               out_ref):
  blk = out_ref.shape[0]
  h = jnp.zeros((blk, H), jnp.float32)
  c = jnp.zeros((blk, H), jnp.float32)
  for t in range(T):
    a = agg_ref[0, t] + agg_ref[1, t] + y2_ref[t]
    h2 = jax.nn.relu(dinv_ref[t] * a + b2_ref[...])
    g = (jnp.dot(h2, wih_ref[...], preferred_element_type=jnp.float32)
         + jnp.dot(h, whh_ref[...], preferred_element_type=jnp.float32)
         + bg_ref[...])
    i_g = jax.nn.sigmoid(g[:, 0 * H:1 * H])
    f_g = jax.nn.sigmoid(g[:, 1 * H:2 * H])
    g_g = jnp.tanh(g[:, 2 * H:3 * H])
    o_g = jax.nn.sigmoid(g[:, 3 * H:4 * H])
    c = f_g * c + i_g * g_g
    h = o_g * jnp.tanh(c)
  out_ref[...] = h


def _tc_lstm(aggp, y2, dinv, b2, wih_t, whh_t, bg, blk=2048):
  return pl.pallas_call(
      _lstm_body,
      grid=(P // blk,),
      in_specs=[
          pl.BlockSpec((NC, T, blk, H), lambda i: (0, 0, i, 0)),
          pl.BlockSpec((T, blk, H), lambda i: (0, i, 0)),
          pl.BlockSpec((T, blk, 1), lambda i: (0, i, 0)),
          pl.BlockSpec((1, H), lambda i: (0, 0)),
          pl.BlockSpec((H, 4 * H), lambda i: (0, 0)),
          pl.BlockSpec((H, 4 * H), lambda i: (0, 0)),
          pl.BlockSpec((1, 4 * H), lambda i: (0, 0)),
      ],
      out_specs=pl.BlockSpec((blk, H), lambda i: (i, 0)),
      out_shape=jax.ShapeDtypeStruct((P, H), jnp.float32),
  )(aggp, y2, dinv, b2, wih_t, whh_t, bg)


def kernel(node_features_seq, edge_indices_seq, W1, b1, W2, b2,
           W_ih, W_hh, b_ih, b_hh):
  x = jnp.pad(node_features_seq, ((0, 0), (0, P - N), (0, 0)))
  pad_src = (jnp.arange(EPAD - E) % N).astype(jnp.int32)
  pad_dst = (N + (jnp.arange(EPAD - E) % (P - N))).astype(jnp.int32)
  src = jnp.concatenate(
      [edge_indices_seq[:, 0, :],
       jnp.broadcast_to(pad_src, (T, EPAD - E))], axis=1)
  dst = jnp.concatenate(
      [edge_indices_seq[:, 1, :],
       jnp.broadcast_to(pad_dst, (T, EPAD - E))], axis=1)
  src2d = src.reshape(-1, B)
  dst2d = dst.reshape(-1, B)
  zeros = jnp.zeros((RT, H), jnp.float32)

  degp = _sc_deg(dst.reshape(-1))
  xw = _tc_xw(x.reshape(TP, F), W1)
  y1, dinv3 = _tc_y1s(degp, xw)
  dinv_tp = dinv3.reshape(1, TP, 1)

  agg1 = _sc_agg(y1, src2d, dst2d, zeros)
  y2 = _tc_y2(agg1.reshape(NC, TP, H), y1, dinv_tp, b1.reshape(1, H), W2)
  agg2 = _sc_agg(y2, src2d, dst2d, zeros)

  h = _tc_lstm(agg2, y2.reshape(T, P, H), dinv3, b2.reshape(1, H),
               W_ih.T, W_hh.T, (b_ih + b_hh).reshape(1, 4 * H))
  return h[:N]
